# Initial kernel scaffold; baseline (speedup 1.0000x reference)
import jax
import jax.numpy as jnp
from jax.experimental import pallas as pl


def kernel(x, edge_index, W, a_src, a_dst):
    raise NotImplementedError("write your pallas kernel here")


# speedup vs baseline: 65.9282x; 65.9282x over previous
import functools

import jax
import jax.numpy as jnp
from jax import lax
from jax.experimental import pallas as pl
from jax.experimental.pallas import tpu as pltpu
from jax.experimental.pallas import tpu_sc as plsc

N_NODES = 10000
N_EDGES = 320000
F_IN = 128
NH = 8
DH = 16
FO = NH * DH

NC = 2
NS = 16
NW = NC * NS

CHUNK = 128
QROWS = 128
NQ = CHUNK // QROWS
NCHUNKS = N_EDGES // CHUNK
CPW = -(-NCHUNKS // NW)
RPT = 624
RPT_LAST = N_NODES - 15 * RPT

_BR = 1000


def _tc_prologue(x, W, Acat):
    def body(x_ref, w_ref, a_ref, h_ref, t_ref):
        h = jnp.dot(x_ref[...], w_ref[...],
                    preferred_element_type=jnp.float32,
                    precision=lax.Precision.HIGHEST)
        h_ref[...] = h
        t_ref[...] = jnp.dot(h, a_ref[...],
                             preferred_element_type=jnp.float32,
                             precision=lax.Precision.HIGHEST)

    return pl.pallas_call(
        body,
        grid=(N_NODES // _BR,),
        in_specs=[
            pl.BlockSpec((_BR, F_IN), lambda i: (i, 0)),
            pl.BlockSpec((F_IN, FO), lambda i: (0, 0)),
            pl.BlockSpec((F_IN, 32), lambda i: (0, 0)),
        ],
        out_specs=[
            pl.BlockSpec((_BR, FO), lambda i: (i, 0)),
            pl.BlockSpec((_BR, 32), lambda i: (i, 0)),
        ],
        out_shape=[
            jax.ShapeDtypeStruct((N_NODES, FO), jnp.float32),
            jax.ShapeDtypeStruct((N_NODES, 32), jnp.float32),
        ],
    )(x, W, Acat)


def _sc_edge_pass(h, tsrc, tdst, src3, dst3, z128, z16):
    mesh = plsc.VectorSubcoreMesh(core_axis_name="c", subcore_axis_name="s")

    @functools.partial(
        pl.kernel,
        out_type=[
            jax.ShapeDtypeStruct((NC, N_NODES, FO), jnp.float32),
            jax.ShapeDtypeStruct((NC, N_NODES, DH), jnp.float32),
        ],
        mesh=mesh,
        scratch_types=[
            pltpu.VMEM_SHARED((N_NODES, FO), jnp.float32),
            pltpu.VMEM_SHARED((N_NODES, DH), jnp.float32),
            pltpu.VMEM((NQ, QROWS), jnp.int32),
            pltpu.VMEM((NQ, QROWS), jnp.int32),
            pltpu.VMEM((CHUNK, DH), jnp.float32),
            pltpu.VMEM((CHUNK, DH), jnp.float32),
            pltpu.VMEM((CHUNK, DH), jnp.float32),
            pltpu.VMEM((CHUNK, FO), jnp.float32),
            pltpu.SemaphoreType.DMA,
        ],
        compiler_params=pltpu.CompilerParams(use_tc_tiling_on_sc=False),
    )
    def k(h_hbm, ts_hbm, td_hbm, src_hbm, dst_hbm, z128_hbm, z16_hbm,
          pout_hbm, sout_hbm,
          acc, sacc, src_v, dst_v, a_s, a_d, p_v, hrows, sem):
        cid = lax.axis_index("c")
        sid = lax.axis_index("s")
        wid = sid * NC + cid
        lo = sid * RPT

        @pl.when(sid < NS - 1)
        def _():
            rows = pl.ds(lo, RPT)
            pltpu.sync_copy(z128_hbm.at[rows], acc.at[rows])
            pltpu.sync_copy(z16_hbm.at[rows], sacc.at[rows])

        @pl.when(sid == NS - 1)
        def _():
            rows = pl.ds(15 * RPT, RPT_LAST)
            pltpu.sync_copy(z128_hbm.at[rows], acc.at[rows])
            pltpu.sync_copy(z16_hbm.at[rows], sacc.at[rows])

        plsc.subcore_barrier()

        def chunk_body(j, carry):
            gcid = wid + NW * j

            @pl.when(gcid < NCHUNKS)
            def _():
                pltpu.sync_copy(src_hbm.at[gcid], src_v)
                pltpu.sync_copy(dst_hbm.at[gcid], dst_v)
                copies = []
                for q in range(NQ):
                    rows = pl.ds(q * QROWS, QROWS)
                    copies.append(
                        pltpu.async_copy(ts_hbm.at[src_v.at[q]], a_s.at[rows], sem))
                    copies.append(
                        pltpu.async_copy(td_hbm.at[dst_v.at[q]], a_d.at[rows], sem))
                    copies.append(
                        pltpu.async_copy(h_hbm.at[src_v.at[q]], hrows.at[rows], sem))
                for c in copies:
                    c.wait()

                def prow(r, c):
                    v = a_s[r] + a_d[r]
                    p_v[r] = jnp.exp(jnp.where(v >= 0.0, v, 0.2 * v))
                    return c

                lax.fori_loop(0, CHUNK, prow, None, unroll=4)

                def srow(e, c):
                    pr = p_v[e]
                    for hh in range(NH):
                        seg = pl.ds(hh * DH, DH)
                        hrows[e, seg] = hrows[e, seg] * pr[hh]
                    return c

                lax.fori_loop(0, CHUNK, srow, None)

                for q in range(NQ):
                    rows = pl.ds(q * QROWS, QROWS)
                    pltpu.sync_copy(p_v.at[rows], sacc.at[dst_v.at[q]], add=True)
                    pltpu.sync_copy(hrows.at[rows], acc.at[dst_v.at[q]], add=True)

            return carry

        lax.fori_loop(0, CPW, chunk_body, None)
        plsc.subcore_barrier()

        @pl.when(sid < NS - 1)
        def _():
            rows = pl.ds(lo, RPT)
            pltpu.sync_copy(acc.at[rows], pout_hbm.at[cid, rows])
            pltpu.sync_copy(sacc.at[rows], sout_hbm.at[cid, rows])

        @pl.when(sid == NS - 1)
        def _():
            rows = pl.ds(15 * RPT, RPT_LAST)
            pltpu.sync_copy(acc.at[rows], pout_hbm.at[cid, rows])
            pltpu.sync_copy(sacc.at[rows], sout_hbm.at[cid, rows])

    return k(h, tsrc, tdst, src3, dst3, z128, z16)


def _tc_epilogue(p0, p1, s0, s1, K16):
    def body(p0_ref, p1_ref, s0_ref, s1_ref, k_ref, o_ref):
        acc = p0_ref[...] + p1_ref[...]
        r = 1.0 / (s0_ref[...] + s1_ref[...] + 1e-10)
        o_ref[...] = acc * jnp.dot(r, k_ref[...],
                                   preferred_element_type=jnp.float32)

    return pl.pallas_call(
        body,
        grid=(N_NODES // _BR,),
        in_specs=[
            pl.BlockSpec((_BR, FO), lambda i: (i, 0)),
            pl.BlockSpec((_BR, FO), lambda i: (i, 0)),
            pl.BlockSpec((_BR, DH), lambda i: (i, 0)),
            pl.BlockSpec((_BR, DH), lambda i: (i, 0)),
            pl.BlockSpec((DH, FO), lambda i: (0, 0)),
        ],
        out_specs=pl.BlockSpec((_BR, FO), lambda i: (i, 0)),
        out_shape=jax.ShapeDtypeStruct((N_NODES, FO), jnp.float32),
    )(p0, p1, s0, s1, K16)


@jax.jit
def kernel(x, edge_index, W, a_src, a_dst):
    f = jnp.float32
    rows = jnp.arange(F_IN)
    cols = rows // DH
    As = jnp.zeros((F_IN, NH), f).at[rows, cols].set(a_src.reshape(-1))
    Ad = jnp.zeros((F_IN, NH), f).at[rows, cols].set(a_dst.reshape(-1))
    zpad = jnp.zeros((F_IN, NH), f)
    Acat = jnp.concatenate([As, zpad, Ad, zpad], axis=1)

    h, T = _tc_prologue(x, W, Acat)
    tsrc = T[:, 0:16]
    tdst = T[:, 16:32]

    src3 = edge_index[0].reshape(NCHUNKS, NQ, QROWS)
    dst3 = edge_index[1].reshape(NCHUNKS, NQ, QROWS)
    z128 = jnp.zeros((N_NODES, FO), f)
    z16 = jnp.zeros((N_NODES, DH), f)

    pout, sout = _sc_edge_pass(h, tsrc, tdst, src3, dst3, z128, z16)

    K8 = jnp.repeat(jnp.eye(NH, dtype=f), DH, axis=1)
    K16 = jnp.concatenate([K8, jnp.zeros((NH, FO), f)], axis=0)
    return _tc_epilogue(pout[0], pout[1], sout[0], sout[1], K16)
